# Initial kernel scaffold; baseline (speedup 1.0000x reference)
import jax
import jax.numpy as jnp
from jax.experimental import pallas as pl


def kernel(features, edge_index, edge_weight, W1, W2, W3, W4, W5, W6, W7, W8, W9, W10, b1, b2, b3, b4, b5, b6, b7, b8, b9, b10):
    raise NotImplementedError("write your pallas kernel here")


# speedup vs baseline: 4.4930x; 4.4930x over previous
import functools

import jax
import jax.numpy as jnp
from jax import lax
from jax.experimental import pallas as pl
from jax.experimental.pallas import tpu as pltpu
from jax.experimental.pallas import tpu_sc as plsc

N = 10000
D = 128
E = 320000
NC = 2
NS = 16
NW = NC * NS
CHUNK = 128
NCHUNKS = E // CHUNK
BASE_CH = NCHUNKS // NW
EXTRA_CH = NCHUNKS - BASE_CH * NW
RPS = 624
TAIL = N - NS * RPS


def _spmm_body(sup_hbm, src_hbm, dst_hbm, ew_hbm, out_hbm,
               acc_sh, idx_v, didx_v, w_v, rows_v, sem):
    c = lax.axis_index("c")
    s = lax.axis_index("s")
    w = s * NC + c

    for j in range(16):
        for cc in range(D // 16):
            rows_v[j, pl.ds(cc * 16, 16)] = jnp.zeros((16,), jnp.float32)
    row0 = s * RPS
    def _zero(t, carry):
        pltpu.sync_copy(rows_v.at[pl.ds(0, 16)], acc_sh.at[pl.ds(row0 + t * 16, 16)])
        return carry
    lax.fori_loop(0, RPS // 16, _zero, 0)

    @pl.when(s == NS - 1)
    def _zero_tail():
        pltpu.sync_copy(rows_v.at[pl.ds(0, TAIL)], acc_sh.at[pl.ds(NS * RPS, TAIL)])
    plsc.subcore_barrier()

    nch = BASE_CH + jnp.where(w < EXTRA_CH, 1, 0)

    def _chunk(t, carry):
        base = (w + t * NW) * CHUNK
        pltpu.sync_copy(src_hbm.at[pl.ds(base, CHUNK)], idx_v)
        pltpu.sync_copy(dst_hbm.at[pl.ds(base, CHUNK)], didx_v)
        pltpu.sync_copy(ew_hbm.at[pl.ds(base, CHUNK)], w_v)
        pltpu.async_copy(sup_hbm.at[idx_v], rows_v, sem).wait()

        def _grp(g, carry2):
            w16 = w_v[pl.ds(g * 16, 16)]
            for j in range(16):
                r = g * 16 + j
                wj = w16[j]
                for cc in range(D // 16):
                    rows_v[r, pl.ds(cc * 16, 16)] = (
                        rows_v[r, pl.ds(cc * 16, 16)] * wj)
            return carry2
        lax.fori_loop(0, CHUNK // 16, _grp, 0)

        pltpu.sync_copy(rows_v, acc_sh.at[didx_v], add=True)
        return carry
    lax.fori_loop(0, nch, _chunk, 0)

    plsc.subcore_barrier()
    pltpu.sync_copy(acc_sh.at[pl.ds(row0, RPS)], out_hbm.at[c, pl.ds(row0, RPS)])

    @pl.when(s == NS - 1)
    def _pub_tail():
        pltpu.sync_copy(acc_sh.at[pl.ds(NS * RPS, TAIL)],
                        out_hbm.at[c, pl.ds(NS * RPS, TAIL)])


_spmm = pl.kernel(
    _spmm_body,
    out_type=jax.ShapeDtypeStruct((NC, N, D), jnp.float32),
    mesh=plsc.VectorSubcoreMesh(core_axis_name="c", subcore_axis_name="s"),
    scratch_types=[
        pltpu.VMEM_SHARED((N, D), jnp.float32),
        pltpu.VMEM((CHUNK,), jnp.int32),
        pltpu.VMEM((CHUNK,), jnp.int32),
        pltpu.VMEM((CHUNK,), jnp.float32),
        pltpu.VMEM((CHUNK, D), jnp.float32),
        pltpu.SemaphoreType.DMA,
    ],
)


def _mm0_body(x_ref, w_ref, o_ref):
    o_ref[...] = jnp.dot(x_ref[...], w_ref[...],
                         preferred_element_type=jnp.float32)


def _mid_body(p_ref, b_ref, w_ref, o_ref):
    h = jnp.maximum(p_ref[0] + p_ref[1] + b_ref[...], 0.0)
    o_ref[...] = jnp.dot(h, w_ref[...], preferred_element_type=jnp.float32)


def _fin_body(p_ref, b_ref, o_ref):
    y = p_ref[0] + p_ref[1] + b_ref[...]
    m = jnp.max(y)
    lse = jnp.log(jnp.sum(jnp.exp(y - m))) + m
    o_ref[...] = y - lse


_mm0 = pl.pallas_call(
    _mm0_body, out_shape=jax.ShapeDtypeStruct((N, D), jnp.float32))
_mid = pl.pallas_call(
    _mid_body, out_shape=jax.ShapeDtypeStruct((N, D), jnp.float32))
_fin = pl.pallas_call(
    _fin_body, out_shape=jax.ShapeDtypeStruct((N, D), jnp.float32))


def kernel(features, edge_index, edge_weight,
           W1, W2, W3, W4, W5, W6, W7, W8, W9, W10,
           b1, b2, b3, b4, b5, b6, b7, b8, b9, b10):
    src = edge_index[0].astype(jnp.int32)
    dst = edge_index[1].astype(jnp.int32)
    ew = edge_weight.astype(jnp.float32)
    Ws = [W1, W2, W3, W4, W5, W6, W7, W8, W9, W10]
    bs = [b1, b2, b3, b4, b5, b6, b7, b8, b9, b10]

    sup = _mm0(features, Ws[0])
    for i in range(1, 10):
        parts = _spmm(sup, src, dst, ew)
        sup = _mid(parts, bs[i - 1].reshape(1, D), Ws[i])
    parts = _spmm(sup, src, dst, ew)
    y = _fin(parts, bs[9].reshape(1, D))
    return y.reshape(-1)


# speedup vs baseline: 10.8980x; 2.4255x over previous
import functools

import jax
import jax.numpy as jnp
from jax import lax
from jax.experimental import pallas as pl
from jax.experimental.pallas import tpu as pltpu
from jax.experimental.pallas import tpu_sc as plsc

N = 10000
D = 128
E = 320000
NC = 2
NS = 16
NW = NC * NS
CHUNK = 128
EPW = E // NW
FULL = EPW // CHUNK
TAILE = EPW - FULL * CHUNK
RPS = 624
TAIL = N - NS * RPS


def _spmm_body(sup_hbm, src_hbm, dst_hbm, ew_hbm, out_hbm,
               acc_sh, sidxf, didx_v, ew_v, rows_v,
               sem_s, sem_r0, sem_r1, sem_m0, sem_m1):
    c = lax.axis_index("c")
    s = lax.axis_index("s")
    w = s * NC + c
    e0 = w * EPW

    d1 = pltpu.async_copy(src_hbm.at[pl.ds(e0, EPW)], sidxf, sem_s)
    pltpu.async_copy(dst_hbm.at[pl.ds(e0, CHUNK)], didx_v.at[0], sem_m0)
    pltpu.async_copy(ew_hbm.at[pl.ds(e0, CHUNK)], ew_v.at[0], sem_m0)

    for j in range(16):
        for cc in range(D // 16):
            rows_v[0, j, pl.ds(cc * 16, 16)] = jnp.zeros((16,), jnp.float32)
    row0 = s * RPS
    def _zero(t, carry):
        pltpu.sync_copy(rows_v.at[0, pl.ds(0, 16)],
                        acc_sh.at[pl.ds(row0 + t * 16, 16)])
        return carry
    lax.fori_loop(0, RPS // 16, _zero, 0)

    @pl.when(s == NS - 1)
    def _zero_tail():
        pltpu.sync_copy(rows_v.at[0, pl.ds(0, TAIL)], acc_sh.at[pl.ds(NS * RPS, TAIL)])

    d1.wait()
    pltpu.async_copy(sup_hbm.at[sidxf.at[pl.ds(0, CHUNK)]], rows_v.at[0], sem_r0)
    plsc.subcore_barrier()

    sem_r = (sem_r0, sem_r1)
    sem_m = (sem_m0, sem_m1)

    def _phase(t, slot, is_tail=False):
        other = 1 - slot

        if not is_tail:
            @pl.when(t + 1 < FULL)
            def _nxt():
                base = e0 + (t + 1) * CHUNK
                pltpu.async_copy(dst_hbm.at[pl.ds(base, CHUNK)],
                                 didx_v.at[other], sem_m[other])
                pltpu.async_copy(ew_hbm.at[pl.ds(base, CHUNK)],
                                 ew_v.at[other], sem_m[other])
                pltpu.async_copy(sup_hbm.at[sidxf.at[pl.ds((t + 1) * CHUNK, CHUNK)]],
                                 rows_v.at[other], sem_r[other])

            @pl.when(t + 1 == FULL)
            def _nxt_tail():
                base = e0 + FULL * CHUNK
                pltpu.async_copy(dst_hbm.at[pl.ds(base, TAILE)],
                                 didx_v.at[other, pl.ds(0, TAILE)], sem_m[other])
                pltpu.async_copy(ew_hbm.at[pl.ds(base, TAILE)],
                                 ew_v.at[other, pl.ds(0, TAILE)], sem_m[other])
                for q in range(TAILE // 16, CHUNK // 16):
                    ew_v[other, pl.ds(q * 16, 16)] = jnp.zeros((16,), jnp.float32)
                pltpu.async_copy(sup_hbm.at[sidxf.at[pl.ds(FULL * CHUNK, TAILE)]],
                                 rows_v.at[other, pl.ds(0, TAILE)], sem_r[other])

        nb = TAILE if is_tail else CHUNK
        pltpu.make_async_copy(dst_hbm.at[pl.ds(0, nb)],
                              didx_v.at[slot, pl.ds(0, nb)], sem_m[slot]).wait()
        pltpu.make_async_copy(ew_hbm.at[pl.ds(0, nb)],
                              ew_v.at[slot, pl.ds(0, nb)], sem_m[slot]).wait()
        pltpu.make_async_copy(sup_hbm.at[pl.ds(0, nb)],
                              rows_v.at[slot, pl.ds(0, nb)], sem_r[slot]).wait()

        def _grp(g, carry2):
            w16 = ew_v[slot, pl.ds(g * 16, 16)]
            for j in range(16):
                r = g * 16 + j
                wj = w16[j]
                for cc in range(D // 16):
                    rows_v[slot, r, pl.ds(cc * 16, 16)] = (
                        rows_v[slot, r, pl.ds(cc * 16, 16)] * wj)
            return carry2
        lax.fori_loop(0, CHUNK // 16, _grp, 0)

        pltpu.sync_copy(rows_v.at[slot], acc_sh.at[didx_v.at[slot]], add=True)

    def _pair(i, carry):
        _phase(2 * i, 0)
        _phase(2 * i + 1, 1)
        return carry
    lax.fori_loop(0, FULL // 2, _pair, 0)
    _phase(FULL, 0, is_tail=True)

    plsc.subcore_barrier()
    pltpu.sync_copy(acc_sh.at[pl.ds(row0, RPS)], out_hbm.at[c, pl.ds(row0, RPS)])

    @pl.when(s == NS - 1)
    def _pub_tail():
        pltpu.sync_copy(acc_sh.at[pl.ds(NS * RPS, TAIL)],
                        out_hbm.at[c, pl.ds(NS * RPS, TAIL)])


_spmm = pl.kernel(
    _spmm_body,
    out_type=jax.ShapeDtypeStruct((NC, N, D), jnp.float32),
    mesh=plsc.VectorSubcoreMesh(core_axis_name="c", subcore_axis_name="s"),
    scratch_types=[
        pltpu.VMEM_SHARED((N, D), jnp.float32),
        pltpu.VMEM((EPW,), jnp.int32),
        pltpu.VMEM((2, CHUNK), jnp.int32),
        pltpu.VMEM((2, CHUNK), jnp.float32),
        pltpu.VMEM((2, CHUNK, D), jnp.float32),
        pltpu.SemaphoreType.DMA,
        pltpu.SemaphoreType.DMA,
        pltpu.SemaphoreType.DMA,
        pltpu.SemaphoreType.DMA,
        pltpu.SemaphoreType.DMA,
    ],
)


def _mm0_body(x_ref, w_ref, o_ref):
    o_ref[...] = jnp.dot(x_ref[...], w_ref[...],
                         preferred_element_type=jnp.float32)


def _mid_body(p_ref, b_ref, w_ref, o_ref):
    h = jnp.maximum(p_ref[0] + p_ref[1] + b_ref[...], 0.0)
    o_ref[...] = jnp.dot(h, w_ref[...], preferred_element_type=jnp.float32)


def _fin_body(p_ref, b_ref, o_ref):
    y = p_ref[0] + p_ref[1] + b_ref[...]
    m = jnp.max(y)
    lse = jnp.log(jnp.sum(jnp.exp(y - m))) + m
    o_ref[...] = y - lse


_mm0 = pl.pallas_call(
    _mm0_body, out_shape=jax.ShapeDtypeStruct((N, D), jnp.float32))
_mid = pl.pallas_call(
    _mid_body, out_shape=jax.ShapeDtypeStruct((N, D), jnp.float32))
_fin = pl.pallas_call(
    _fin_body, out_shape=jax.ShapeDtypeStruct((N, D), jnp.float32))


def kernel(features, edge_index, edge_weight,
           W1, W2, W3, W4, W5, W6, W7, W8, W9, W10,
           b1, b2, b3, b4, b5, b6, b7, b8, b9, b10):
    src = edge_index[0].astype(jnp.int32)
    dst = edge_index[1].astype(jnp.int32)
    ew = edge_weight.astype(jnp.float32)
    Ws = [W1, W2, W3, W4, W5, W6, W7, W8, W9, W10]
    bs = [b1, b2, b3, b4, b5, b6, b7, b8, b9, b10]

    sup = _mm0(features, Ws[0])
    for i in range(1, 10):
        parts = _spmm(sup, src, dst, ew)
        sup = _mid(parts, bs[i - 1].reshape(1, D), Ws[i])
    parts = _spmm(sup, src, dst, ew)
    y = _fin(parts, bs[9].reshape(1, D))
    return y.reshape(-1)
